# baseline (speedup 1.0000x reference)
import functools

import jax
import jax.numpy as jnp
from jax import lax
from jax.experimental import pallas as pl
from jax.experimental.pallas import tpu as pltpu
from jax.experimental.pallas import tpu_sc as plsc

NC = 2
NS = 16
LANES = 16
NW = NC * NS

CH = 128
D = 128


PG = 8


def _sc_aggregate(np_rows, pages0, pages1):
  rows_per_tile = np_rows // NS
  zcopies = rows_per_tile // 16
  mesh = plsc.VectorSubcoreMesh(core_axis_name="c", subcore_axis_name="s")

  out_type = jax.ShapeDtypeStruct((NC, np_rows, D), jnp.float32)
  scratch = [
      pltpu.VMEM_SHARED((np_rows, D), jnp.float32),
      pltpu.VMEM((PG, CH), jnp.int32),
      pltpu.VMEM((PG, CH), jnp.int32),
      pltpu.VMEM((CH, D), jnp.float32),
      pltpu.VMEM((CH, D), jnp.float32),
      pltpu.VMEM((16, D), jnp.float32),
      pltpu.SemaphoreType.DMA,
      pltpu.SemaphoreType.DMA,
  ]

  def body(x_hbm, srcs_hbm, dsts_hbm, out_hbm, acc_sh, src_page, dst_page,
           rows0, rows1, zrow, sem0, sem1):
    c = lax.axis_index("c")
    s = lax.axis_index("s")
    row_base = s * rows_per_tile
    my_pages = jnp.where(c == 0, pages0, pages1)
    start = jnp.where(c == 0, s * pages0, NS * pages0 + s * pages1)

    zv = jnp.zeros((LANES,), jnp.float32)
    for r in range(16):
      for cc in range(D // LANES):
        zrow[r, pl.ds(cc * LANES, LANES)] = zv

    def zbody(k, carry):
      pltpu.sync_copy(zrow, acc_sh.at[pl.ds(row_base + k * 16, 16), :])
      return carry
    lax.fori_loop(0, zcopies, zbody, 0)

    plsc.subcore_barrier()

    rows = (rows0, rows1)
    sems = (sem0, sem1)

    def page(p, carry):
      pltpu.sync_copy(srcs_hbm.at[start + p], src_page)
      pltpu.sync_copy(dsts_hbm.at[start + p], dst_page)
      pltpu.async_copy(x_hbm.at[src_page.at[0]], rows[0], sems[0])
      for k in range(PG):
        if k + 1 < PG:
          pltpu.async_copy(x_hbm.at[src_page.at[k + 1]], rows[(k + 1) % 2],
                           sems[(k + 1) % 2])
        pltpu.make_async_copy(x_hbm.at[src_page.at[k]], rows[k % 2],
                              sems[k % 2]).wait()
        pltpu.sync_copy(rows[k % 2], acc_sh.at[dst_page.at[k]], add=True)
      return carry
    lax.fori_loop(0, my_pages, page, 0)

    plsc.subcore_barrier()
    pltpu.sync_copy(acc_sh.at[pl.ds(row_base, rows_per_tile), :],
                    out_hbm.at[c, pl.ds(row_base, rows_per_tile), :])

  return pl.kernel(body, out_type=out_type, mesh=mesh,
                   scratch_types=scratch)


def _sc_counts(np_rows, n_pages):
  rows_per_tile = np_rows // NS
  zcopies = rows_per_tile // 16
  mesh = plsc.VectorSubcoreMesh(core_axis_name="c", subcore_axis_name="s")

  out_type = jax.ShapeDtypeStruct((NC, np_rows, D), jnp.float32)
  scratch = [
      pltpu.VMEM_SHARED((np_rows, D), jnp.float32),
      pltpu.VMEM((PG, CH), jnp.int32),
      pltpu.VMEM((CH, D), jnp.float32),
      pltpu.VMEM((16, D), jnp.float32),
  ]

  def body(dsts_hbm, out_hbm, acc_sh, dst_page, ones_v, zrow):
    c = lax.axis_index("c")
    s = lax.axis_index("s")
    w = s * NC + c
    row_base = s * rows_per_tile

    zv = jnp.zeros((LANES,), jnp.float32)
    for r in range(16):
      for cc in range(D // LANES):
        zrow[r, pl.ds(cc * LANES, LANES)] = zv
    ov = jnp.ones((LANES,), jnp.float32)
    def obody(r, carry):
      for cc in range(D // LANES):
        ones_v[r, pl.ds(cc * LANES, LANES)] = ov
      return carry
    lax.fori_loop(0, CH, obody, 0)

    def zbody(k, carry):
      pltpu.sync_copy(zrow, acc_sh.at[pl.ds(row_base + k * 16, 16), :])
      return carry
    lax.fori_loop(0, zcopies, zbody, 0)

    plsc.subcore_barrier()

    def page(p, carry):
      pltpu.sync_copy(dsts_hbm.at[w * (n_pages // NW) + p], dst_page)
      for k in range(PG):
        pltpu.sync_copy(ones_v, acc_sh.at[dst_page.at[k]], add=True)
      return carry
    lax.fori_loop(0, n_pages // NW, page, 0)

    plsc.subcore_barrier()
    pltpu.sync_copy(acc_sh.at[pl.ds(row_base, rows_per_tile), :],
                    out_hbm.at[c, pl.ds(row_base, rows_per_tile), :])

  return pl.kernel(body, out_type=out_type, mesh=mesh,
                   scratch_types=scratch)


def _tc_combine(p, cnt, x, Wl, Wr, b, elu):
  nrows = x.shape[0]
  bn = 256
  grid = (nrows // bn,)
  b2 = b.reshape(1, D)

  def body(p_ref, c_ref, x_ref, wl_ref, wr_ref, b_ref, o_ref):
    agg_sum = p_ref[0] + p_ref[1]
    cnt_col = c_ref[0][:, 0:1] + c_ref[1][:, 0:1]
    agg = agg_sum / jnp.maximum(cnt_col, 1.0)
    out = (jnp.dot(agg, wl_ref[:], preferred_element_type=jnp.float32)
           + jnp.dot(x_ref[:], wr_ref[:], preferred_element_type=jnp.float32)
           + b_ref[:])
    nrm = jnp.sqrt(jnp.sum(out * out, axis=1, keepdims=True))
    out = out / jnp.maximum(nrm, 1e-12)
    if elu:
      out = jnp.where(out > 0, out, jnp.exp(jnp.minimum(out, 0.0)) - 1.0)
    o_ref[:] = out

  return pl.pallas_call(
      body,
      grid=grid,
      in_specs=[
          pl.BlockSpec((NC, bn, D), lambda i: (0, i, 0)),
          pl.BlockSpec((NC, bn, D), lambda i: (0, i, 0)),
          pl.BlockSpec((bn, D), lambda i: (i, 0)),
          pl.BlockSpec((D, D), lambda i: (0, 0)),
          pl.BlockSpec((D, D), lambda i: (0, 0)),
          pl.BlockSpec((1, D), lambda i: (0, 0)),
      ],
      out_specs=pl.BlockSpec((bn, D), lambda i: (i, 0)),
      out_shape=jax.ShapeDtypeStruct((nrows, D), jnp.float32),
  )(p, cnt, x, Wl, Wr, b2)


def kernel(x, edge_index, Wl0, Wr0, b0, Wl1, Wr1, b1, Wl2, Wr2, b2):
  n, d = x.shape
  assert d == D
  e = edge_index.shape[1]

  bn = 256
  np_rows = ((n + 1 + bn - 1) // bn) * bn

  grain = CH * PG
  t_pages = ((e + grain - 1) // grain + NW - 1) // NW * NW
  e_pad = t_pages * grain

  src = edge_index[0]
  dst = edge_index[1]
  pad_e = e_pad - e
  if pad_e:
    src = jnp.concatenate([src, jnp.zeros((pad_e,), jnp.int32)])
    dst = jnp.concatenate([dst, jnp.full((pad_e,), n, jnp.int32)])
  srcs = src.reshape(t_pages, PG, CH)
  dsts = dst.reshape(t_pages, PG, CH)

  pages_tot = t_pages // NS
  pages0 = max(1, min(pages_tot, round(pages_tot * 1.0)))
  pages1 = pages_tot - pages0

  xp = jnp.pad(x, ((0, np_rows - n), (0, 0)))

  sc_aggr = _sc_aggregate(np_rows, pages0, pages1)
  cnt = _sc_counts(np_rows, t_pages)(dsts)

  h = xp
  p = sc_aggr(h, srcs, dsts)
  h = _tc_combine(p, cnt, h, Wl0, Wr0, b0, elu=True)
  p = sc_aggr(h, srcs, dsts)
  h = _tc_combine(p, cnt, h, Wl1, Wr1, b1, elu=True)
  p = sc_aggr(h, srcs, dsts)
  h = _tc_combine(p, cnt, h, Wl2, Wr2, b2, elu=False)
  return h[:n]


# speedup vs baseline: 1.5504x; 1.5504x over previous
import functools

import jax
import jax.numpy as jnp
from jax import lax
from jax.experimental import pallas as pl
from jax.experimental.pallas import tpu as pltpu
from jax.experimental.pallas import tpu_sc as plsc

NC = 2
NS = 16
LANES = 16
NW = NC * NS

CH = 128
D = 128


PG = 16


def _sc_aggregate(np_rows, pages0, pages1):
  rows_per_tile = np_rows // NS
  zcopies = rows_per_tile // 16
  mesh = plsc.VectorSubcoreMesh(core_axis_name="c", subcore_axis_name="s")

  out_type = jax.ShapeDtypeStruct((NC, np_rows, D), jnp.float32)
  scratch = [
      pltpu.VMEM_SHARED((np_rows, D), jnp.float32),
      pltpu.VMEM((PG, CH), jnp.int32),
      pltpu.VMEM((PG, CH), jnp.int32),
      pltpu.VMEM((CH, D), jnp.float32),
      pltpu.VMEM((CH, D), jnp.float32),
      pltpu.VMEM((16, D), jnp.float32),
      pltpu.SemaphoreType.DMA,
      pltpu.SemaphoreType.DMA,
      pltpu.SemaphoreType.DMA,
      pltpu.SemaphoreType.DMA,
  ]

  def body(x_hbm, srcs_hbm, dsts_hbm, out_hbm, acc_sh, src_page, dst_page,
           rows0, rows1, zrow, gsem0, gsem1, ssem0, ssem1):
    c = lax.axis_index("c")
    s = lax.axis_index("s")
    row_base = s * rows_per_tile
    my_pages = jnp.where(c == 0, pages0, pages1)
    start = jnp.where(c == 0, s * pages0, NS * pages0 + s * pages1)

    zv = jnp.zeros((LANES,), jnp.float32)
    for r in range(16):
      for cc in range(D // LANES):
        zrow[r, pl.ds(cc * LANES, LANES)] = zv

    def zbody(k, carry):
      pltpu.sync_copy(zrow, acc_sh.at[pl.ds(row_base + k * 16, 16), :])
      return carry
    lax.fori_loop(0, zcopies, zbody, 0)

    plsc.subcore_barrier()

    rows = (rows0, rows1)
    gsems = (gsem0, gsem1)
    ssems = (ssem0, ssem1)

    def page(p, carry):
      pltpu.sync_copy(srcs_hbm.at[start + p], src_page)
      pltpu.sync_copy(dsts_hbm.at[start + p], dst_page)
      pltpu.async_copy(x_hbm.at[src_page.at[0]], rows[0], gsems[0])
      for k in range(PG):
        pltpu.make_async_copy(x_hbm.at[src_page.at[k]], rows[k % 2],
                              gsems[k % 2]).wait()
        pltpu.async_copy(rows[k % 2], acc_sh.at[dst_page.at[k]],
                         ssems[k % 2], add=True)
        if k >= 1:
          pltpu.make_async_copy(rows[(k - 1) % 2],
                                acc_sh.at[dst_page.at[k - 1]],
                                ssems[(k - 1) % 2]).wait()
        if k + 1 < PG:
          pltpu.async_copy(x_hbm.at[src_page.at[k + 1]], rows[(k + 1) % 2],
                           gsems[(k + 1) % 2])
      pltpu.make_async_copy(rows[(PG - 1) % 2],
                            acc_sh.at[dst_page.at[PG - 1]],
                            ssems[(PG - 1) % 2]).wait()
      return carry
    lax.fori_loop(0, my_pages, page, 0)

    plsc.subcore_barrier()
    pltpu.sync_copy(acc_sh.at[pl.ds(row_base, rows_per_tile), :],
                    out_hbm.at[c, pl.ds(row_base, rows_per_tile), :])

  return pl.kernel(body, out_type=out_type, mesh=mesh,
                   scratch_types=scratch)


def _sc_counts(np_rows, n_pages):
  rows_per_tile = np_rows // NS
  zcopies = rows_per_tile // 16
  mesh = plsc.VectorSubcoreMesh(core_axis_name="c", subcore_axis_name="s")

  out_type = jax.ShapeDtypeStruct((NC, np_rows, D), jnp.float32)
  scratch = [
      pltpu.VMEM_SHARED((np_rows, D), jnp.float32),
      pltpu.VMEM((PG, CH), jnp.int32),
      pltpu.VMEM((CH, D), jnp.float32),
      pltpu.VMEM((16, D), jnp.float32),
      pltpu.SemaphoreType.DMA,
  ]

  def body(dsts_hbm, out_hbm, acc_sh, dst_page, ones_v, zrow, ssem):
    c = lax.axis_index("c")
    s = lax.axis_index("s")
    w = s * NC + c
    row_base = s * rows_per_tile

    zv = jnp.zeros((LANES,), jnp.float32)
    for r in range(16):
      for cc in range(D // LANES):
        zrow[r, pl.ds(cc * LANES, LANES)] = zv
    ov = jnp.ones((LANES,), jnp.float32)
    def obody(r, carry):
      for cc in range(D // LANES):
        ones_v[r, pl.ds(cc * LANES, LANES)] = ov
      return carry
    lax.fori_loop(0, CH, obody, 0)

    def zbody(k, carry):
      pltpu.sync_copy(zrow, acc_sh.at[pl.ds(row_base + k * 16, 16), :])
      return carry
    lax.fori_loop(0, zcopies, zbody, 0)

    plsc.subcore_barrier()

    def page(p, carry):
      pltpu.sync_copy(dsts_hbm.at[w * (n_pages // NW) + p], dst_page)
      for k in range(PG):
        pltpu.async_copy(ones_v, acc_sh.at[dst_page.at[k]], ssem, add=True)
      for k in range(PG):
        pltpu.make_async_copy(ones_v, acc_sh.at[dst_page.at[k]],
                              ssem).wait()
      return carry
    lax.fori_loop(0, n_pages // NW, page, 0)

    plsc.subcore_barrier()
    pltpu.sync_copy(acc_sh.at[pl.ds(row_base, rows_per_tile), :],
                    out_hbm.at[c, pl.ds(row_base, rows_per_tile), :])

  return pl.kernel(body, out_type=out_type, mesh=mesh,
                   scratch_types=scratch)


def _tc_combine(p, cnt, x, Wl, Wr, b, elu):
  nrows = x.shape[0]
  bn = 256
  grid = (nrows // bn,)
  b2 = b.reshape(1, D)

  def body(p_ref, c_ref, x_ref, wl_ref, wr_ref, b_ref, o_ref):
    agg_sum = p_ref[0] + p_ref[1]
    cnt_col = c_ref[0][:, 0:1] + c_ref[1][:, 0:1]
    agg = agg_sum / jnp.maximum(cnt_col, 1.0)
    out = (jnp.dot(agg, wl_ref[:], preferred_element_type=jnp.float32)
           + jnp.dot(x_ref[:], wr_ref[:], preferred_element_type=jnp.float32)
           + b_ref[:])
    nrm = jnp.sqrt(jnp.sum(out * out, axis=1, keepdims=True))
    out = out / jnp.maximum(nrm, 1e-12)
    if elu:
      out = jnp.where(out > 0, out, jnp.exp(jnp.minimum(out, 0.0)) - 1.0)
    o_ref[:] = out

  return pl.pallas_call(
      body,
      grid=grid,
      in_specs=[
          pl.BlockSpec((NC, bn, D), lambda i: (0, i, 0)),
          pl.BlockSpec((NC, bn, D), lambda i: (0, i, 0)),
          pl.BlockSpec((bn, D), lambda i: (i, 0)),
          pl.BlockSpec((D, D), lambda i: (0, 0)),
          pl.BlockSpec((D, D), lambda i: (0, 0)),
          pl.BlockSpec((1, D), lambda i: (0, 0)),
      ],
      out_specs=pl.BlockSpec((bn, D), lambda i: (i, 0)),
      out_shape=jax.ShapeDtypeStruct((nrows, D), jnp.float32),
  )(p, cnt, x, Wl, Wr, b2)


def kernel(x, edge_index, Wl0, Wr0, b0, Wl1, Wr1, b1, Wl2, Wr2, b2):
  n, d = x.shape
  assert d == D
  e = edge_index.shape[1]

  bn = 256
  np_rows = ((n + 1 + bn - 1) // bn) * bn

  grain = CH * PG
  t_pages = ((e + grain - 1) // grain + NW - 1) // NW * NW
  e_pad = t_pages * grain

  src = edge_index[0]
  dst = edge_index[1]
  pad_e = e_pad - e
  if pad_e:
    src = jnp.concatenate([src, jnp.zeros((pad_e,), jnp.int32)])
    dst = jnp.concatenate([dst, jnp.full((pad_e,), n, jnp.int32)])
  srcs = src.reshape(t_pages, PG, CH)
  dsts = dst.reshape(t_pages, PG, CH)

  pages_tot = t_pages // NS
  pages0 = max(1, min(pages_tot - 1, round(pages_tot * 0.9)))
  pages1 = pages_tot - pages0

  xp = jnp.pad(x, ((0, np_rows - n), (0, 0)))

  sc_aggr = _sc_aggregate(np_rows, pages0, pages1)
  cnt = _sc_counts(np_rows, t_pages)(dsts)

  h = xp
  p = sc_aggr(h, srcs, dsts)
  h = _tc_combine(p, cnt, h, Wl0, Wr0, b0, elu=True)
  p = sc_aggr(h, srcs, dsts)
  h = _tc_combine(p, cnt, h, Wl1, Wr1, b1, elu=True)
  p = sc_aggr(h, srcs, dsts)
  h = _tc_combine(p, cnt, h, Wl2, Wr2, b2, elu=False)
  return h[:n]
